# baseline (speedup 1.0000x reference)
import functools

import jax
import jax.numpy as jnp
from jax import lax
from jax.experimental import pallas as pl
from jax.experimental.pallas import tpu as pltpu
from jax.experimental.pallas import tpu_sc as plsc

NC = 2
NS = 16
CH = 128
NBUF = 3
F0 = 0.615


def _split(e):
    tot_ch = -(-e // CH)
    t0 = F0 * tot_ch / NS
    k0 = max(5, min((v for v in range(int(t0) - 3, int(t0) + 5) if v % 3 == 2),
                    key=lambda v: abs(v - t0)))
    k1m = max(5, -(-(tot_ch - NS * k0) // NS))
    k1 = k1m + (2 - k1m) % 3
    return k0, k1


def _core_k_off(k0, k1):
    c = lax.axis_index("c")
    s = lax.axis_index("s")
    kd = jnp.where(c == 0, k0, k1)
    off = jnp.where(c == 0, s * k0, NS * k0 + s * k1)
    return c, s, kd, off


def _deg_kernel(n_pad, k0, k1):
    npt = n_pad // NS
    kmax = max(k0, k1)

    @functools.partial(
        pl.kernel,
        out_type=jax.ShapeDtypeStruct((NC, n_pad), jnp.float32),
        mesh=plsc.VectorSubcoreMesh(core_axis_name="c", subcore_axis_name="s"),
        scratch_types=[
            pltpu.VMEM((kmax, CH), jnp.int32),
            pltpu.VMEM((kmax, CH), jnp.float32),
            pltpu.VMEM((npt,), jnp.float32),
            pltpu.VMEM_SHARED((n_pad,), jnp.float32),
        ],
        compiler_params=pltpu.CompilerParams(needs_layout_passes=False,
                                             use_tc_tiling_on_sc=False),
    )
    def k(dst_hbm, ew_hbm, out_hbm, dstv, ewv, zb, deg_sh):
        c, s, kd, off = _core_k_off(k0, k1)

        def zfill(i, _):
            zb[pl.ds(i * 16, 16)] = jnp.zeros((16,), jnp.float32)
            return 0

        lax.fori_loop(0, npt // 16, zfill, 0)
        pltpu.sync_copy(zb, deg_sh.at[pl.ds(s * npt, npt)])
        pltpu.sync_copy(dst_hbm.at[pl.ds(off, kmax)], dstv)
        pltpu.sync_copy(ew_hbm.at[pl.ds(off, kmax)], ewv)

        def chunk(j, _):
            pltpu.sync_copy(ewv.at[j], deg_sh.at[dstv.at[j]], add=True)
            return 0

        lax.fori_loop(0, kd, chunk, 0)
        plsc.subcore_barrier()
        pltpu.sync_copy(deg_sh.at[pl.ds(s * npt, npt)],
                        out_hbm.at[c, pl.ds(s * npt, npt)])

    return k


def _msg_kernel(n_pad, k0, k1, h):
    npt = n_pad // NS
    zr = 64
    kmax = max(k0, k1)

    @functools.partial(
        pl.kernel,
        out_type=jax.ShapeDtypeStruct((NC, n_pad, h), jnp.float32),
        mesh=plsc.VectorSubcoreMesh(core_axis_name="c", subcore_axis_name="s"),
        scratch_types=[
            pltpu.VMEM((kmax + 2, CH), jnp.int32),
            pltpu.VMEM((kmax, CH), jnp.int32),
            pltpu.VMEM((kmax * CH,), jnp.float32),
            pltpu.VMEM((NBUF, CH, h), jnp.bfloat16),
            pltpu.VMEM((NBUF, CH, h), jnp.float32),
            pltpu.VMEM((zr, h), jnp.float32),
            pltpu.VMEM_SHARED((n_pad, h), jnp.float32),
            [pltpu.SemaphoreType.DMA] * NBUF,
            [pltpu.SemaphoreType.DMA] * NBUF,
        ],
        compiler_params=pltpu.CompilerParams(needs_layout_passes=False,
                                             use_tc_tiling_on_sc=False),
    )
    def k(src_hbm, dst_hbm, ewf_hbm, h2_hbm, out_hbm,
          srcv, dstv, ewv, rowsb, rowsv, zb, acc_sh, gsems, ssems):
        c, s, kd, off = _core_k_off(k0, k1)
        base = s * npt

        def zfill(i, _):
            for q in range(h // 16):
                zb[i, pl.ds(q * 16, 16)] = jnp.zeros((16,), jnp.float32)
            return 0

        lax.fori_loop(0, zr, zfill, 0)
        for i in range(npt // zr):
            pltpu.sync_copy(zb, acc_sh.at[pl.ds(base + i * zr, zr)])

        pltpu.sync_copy(src_hbm.at[pl.ds(off, kmax + 2)], srcv)
        pltpu.sync_copy(dst_hbm.at[pl.ds(off, kmax)], dstv)
        pltpu.sync_copy(ewf_hbm.at[pl.ds(off * CH, kmax * CH)], ewv)

        def issue_gather(j, b):
            pltpu.async_copy(h2_hbm.at[srcv.at[j]], rowsb.at[b], gsems[b])

        def wait_gather(j, b):
            pltpu.make_async_copy(h2_hbm.at[srcv.at[j]], rowsb.at[b],
                                  gsems[b]).wait()

        def issue_scatter(j, b):
            pltpu.async_copy(rowsv.at[b], acc_sh.at[dstv.at[j]], ssems[b],
                             add=True)

        def wait_scatter(j, b):
            pltpu.make_async_copy(rowsv.at[b], acc_sh.at[dstv.at[j]],
                                  ssems[b]).wait()

        def scale(j, b):
            def grp(g, _):
                for kk in range(16):
                    le = g * 16 + kk
                    bi = jnp.full((16,), j * CH + le, jnp.int32)
                    wv = plsc.load_gather(ewv, [bi])
                    for q in range(h // 32):
                        mb = rowsb[b, le, pl.ds(q * 32, 32)]
                        va, vb = plsc.unpack(
                            mb, format=plsc.PackFormat.INTERLEAVED)
                        rowsv[b, le, pl.ds(q * 32, 16)] = va * wv
                        rowsv[b, le, pl.ds(q * 32 + 16, 16)] = vb * wv
                return 0

            lax.fori_loop(0, CH // 16, grp, 0)

        issue_gather(0, 0)
        issue_gather(1, 1)
        wait_gather(0, 0)
        scale(0, 0)
        issue_scatter(0, 0)
        issue_gather(2, 2)
        wait_gather(1, 1)
        scale(1, 1)
        issue_scatter(1, 1)
        wait_scatter(0, 0)
        issue_gather(3, 0)

        def body(it, _):
            jj = 2 + it * NBUF
            for db in range(NBUF):
                j = jj + db
                b = (2 + db) % NBUF
                wait_gather(j, b)
                scale(j, b)
                issue_scatter(j, b)
                bn = (b + 2) % NBUF
                wait_scatter(j - 1, bn)
                issue_gather(j + 2, bn)
            return 0

        lax.fori_loop(0, (kd - 2) // NBUF, body, 0)
        wait_gather(kd, 2)
        wait_gather(kd + 1, 0)
        wait_scatter(kd - 1, 1)

        plsc.subcore_barrier()
        for i in range(npt // zr):
            pltpu.sync_copy(acc_sh.at[pl.ds(base + i * zr, zr)],
                            out_hbm.at[c, pl.ds(base + i * zr, zr)])

    return k


def _h2_body(x_ref, w1_ref, m1_ref, deg_ref, h2_ref, h2b_ref):
    wm = w1_ref[...] * m1_ref[...]
    hh = jnp.dot(x_ref[...], wm, preferred_element_type=jnp.float32)
    d = deg_ref[0, :] + deg_ref[1, :] + 1.0
    a = lax.rsqrt(d)
    h2 = hh * a[:, None]
    h2_ref[...] = h2
    rb = h2.shape[0]
    hgrp = h2.shape[1] // 32
    h2p = jnp.swapaxes(h2.reshape(rb, hgrp, 2, 16), 2, 3).reshape(rb, -1)
    h2b_ref[...] = h2p.astype(jnp.bfloat16)


def _final_body(acc_ref, h2_ref, deg_ref, b1_ref, w2_ref, m2_ref, b2_ref,
                out_ref):
    d = deg_ref[0, :] + deg_ref[1, :] + 1.0
    a = lax.rsqrt(d)
    tot = acc_ref[0] + acc_ref[1] + h2_ref[...]
    agg = tot * a[:, None] + b1_ref[0][None, :]
    hr = jnp.maximum(agg, 0.0)
    out_ref[...] = (jnp.dot(hr, w2_ref[...] * m2_ref[...],
                            preferred_element_type=jnp.float32)
                    + b2_ref[0][None, :])


def kernel(x, edge_index, edge_weight, W1, W1_mask, b1, W2, W2_mask, b2):
    n, f_in = x.shape
    h = W1.shape[1]
    c_out = W2.shape[1]
    e = edge_weight.shape[0]

    n_pad = ((n + NS * 16 - 1) // (NS * 16)) * (NS * 16)
    k0, k1 = _split(e)
    tot = NS * k0 + (NS - 1) * k1 + max(k0, k1) + 2
    e_pad = tot * CH

    src = edge_index[0].astype(jnp.int32)
    dst = edge_index[1].astype(jnp.int32)
    pad = e_pad - e
    src2 = jnp.concatenate([src, jnp.zeros((pad,), jnp.int32)]).reshape(tot, CH)
    dst2 = jnp.concatenate([dst, jnp.zeros((pad,), jnp.int32)]).reshape(tot, CH)
    ewp = jnp.concatenate([edge_weight, jnp.zeros((pad,), jnp.float32)])
    ew2 = ewp.reshape(tot, CH)

    deg2 = _deg_kernel(n_pad, k0, k1)(dst2, ew2)

    rb = 1024
    grid = -(-n // rb)
    h2 = pl.pallas_call(
        _h2_body,
        grid=(grid,),
        in_specs=[
            pl.BlockSpec((rb, f_in), lambda j: (j, 0)),
            pl.BlockSpec((f_in, h), lambda j: (0, 0)),
            pl.BlockSpec((f_in, h), lambda j: (0, 0)),
            pl.BlockSpec((NC, rb), lambda j: (0, j)),
        ],
        out_specs=[pl.BlockSpec((rb, h), lambda j: (j, 0)),
                   pl.BlockSpec((rb, h), lambda j: (j, 0))],
        out_shape=[jax.ShapeDtypeStruct((n, h), jnp.float32),
                   jax.ShapeDtypeStruct((n, h), jnp.bfloat16)],
    )(x, W1, W1_mask, deg2)
    h2, h2b = h2

    acc = _msg_kernel(n_pad, k0, k1, h)(src2, dst2, ewp, h2b)

    logits = pl.pallas_call(
        _final_body,
        grid=(grid,),
        in_specs=[
            pl.BlockSpec((NC, rb, h), lambda j: (0, j, 0)),
            pl.BlockSpec((rb, h), lambda j: (j, 0)),
            pl.BlockSpec((NC, rb), lambda j: (0, j)),
            pl.BlockSpec((1, h), lambda j: (0, 0)),
            pl.BlockSpec((h, c_out), lambda j: (0, 0)),
            pl.BlockSpec((h, c_out), lambda j: (0, 0)),
            pl.BlockSpec((1, c_out), lambda j: (0, 0)),
        ],
        out_specs=pl.BlockSpec((rb, c_out), lambda j: (j, 0)),
        out_shape=jax.ShapeDtypeStruct((n, c_out), jnp.float32),
    )(acc, h2, deg2, b1.reshape(1, h), W2, W2_mask, b2.reshape(1, c_out))

    return logits


# speedup vs baseline: 1.2869x; 1.2869x over previous
import functools

import jax
import jax.numpy as jnp
from jax import lax
from jax.experimental import pallas as pl
from jax.experimental.pallas import tpu as pltpu
from jax.experimental.pallas import tpu_sc as plsc

NC = 2
NS = 16
CH = 128
NBUF = 3
F0 = 0.615


def _split(e):
    tot_ch = -(-e // CH)
    t0 = F0 * tot_ch / NS
    k0 = max(5, min((v for v in range(int(t0) - 3, int(t0) + 5) if v % 3 == 2),
                    key=lambda v: abs(v - t0)))
    k1m = max(5, -(-(tot_ch - NS * k0) // NS))
    k1 = k1m + (2 - k1m) % 3
    return k0, k1


def _core_k_off(k0, k1):
    c = lax.axis_index("c")
    s = lax.axis_index("s")
    kd = jnp.where(c == 0, k0, k1)
    off = jnp.where(c == 0, s * k0, NS * k0 + s * k1)
    return c, s, kd, off


def _deg_kernel(n_pad, k0, k1):
    npt = n_pad // NS
    kmax = max(k0, k1)

    @functools.partial(
        pl.kernel,
        out_type=jax.ShapeDtypeStruct((NC, n_pad), jnp.float32),
        mesh=plsc.VectorSubcoreMesh(core_axis_name="c", subcore_axis_name="s"),
        scratch_types=[
            pltpu.VMEM((kmax, CH), jnp.int32),
            pltpu.VMEM((kmax, CH), jnp.float32),
            pltpu.VMEM((npt,), jnp.float32),
            pltpu.VMEM_SHARED((n_pad,), jnp.float32),
        ],
        compiler_params=pltpu.CompilerParams(needs_layout_passes=False,
                                             use_tc_tiling_on_sc=False),
    )
    def k(dst_hbm, ew_hbm, out_hbm, dstv, ewv, zb, deg_sh):
        c, s, kd, off = _core_k_off(k0, k1)

        def zfill(i, _):
            zb[pl.ds(i * 16, 16)] = jnp.zeros((16,), jnp.float32)
            return 0

        lax.fori_loop(0, npt // 16, zfill, 0)
        pltpu.sync_copy(zb, deg_sh.at[pl.ds(s * npt, npt)])
        pltpu.sync_copy(dst_hbm.at[pl.ds(off, kmax)], dstv)
        pltpu.sync_copy(ew_hbm.at[pl.ds(off, kmax)], ewv)

        def chunk(j, _):
            pltpu.sync_copy(ewv.at[j], deg_sh.at[dstv.at[j]], add=True)
            return 0

        lax.fori_loop(0, kd, chunk, 0)
        plsc.subcore_barrier()
        pltpu.sync_copy(deg_sh.at[pl.ds(s * npt, npt)],
                        out_hbm.at[c, pl.ds(s * npt, npt)])

    return k


def _msg_kernel(n_pad, k0, k1, h):
    npt = n_pad // NS
    zr = 64
    kmax = max(k0, k1)

    @functools.partial(
        pl.kernel,
        out_type=jax.ShapeDtypeStruct((NC, n_pad, h), jnp.float32),
        mesh=plsc.VectorSubcoreMesh(core_axis_name="c", subcore_axis_name="s"),
        scratch_types=[
            pltpu.VMEM((kmax + 2, CH), jnp.int32),
            pltpu.VMEM((kmax, CH), jnp.int32),
            pltpu.VMEM((kmax * CH,), jnp.float32),
            pltpu.VMEM((NBUF, CH, h), jnp.bfloat16),
            pltpu.VMEM((NBUF, CH, h), jnp.float32),
            pltpu.VMEM((zr, h), jnp.float32),
            pltpu.VMEM_SHARED((n_pad, h), jnp.float32),
            [pltpu.SemaphoreType.DMA] * NBUF,
            [pltpu.SemaphoreType.DMA] * NBUF,
        ],
        compiler_params=pltpu.CompilerParams(needs_layout_passes=False,
                                             use_tc_tiling_on_sc=False),
    )
    def k(src_hbm, dst_hbm, ewf_hbm, h2_hbm, out_hbm,
          srcv, dstv, ewv, rowsb, rowsv, zb, acc_sh, gsems, ssems):
        c, s, kd, off = _core_k_off(k0, k1)
        base = s * npt

        def zfill(i, _):
            for q in range(h // 16):
                zb[i, pl.ds(q * 16, 16)] = jnp.zeros((16,), jnp.float32)
            return 0

        lax.fori_loop(0, zr, zfill, 0)
        for i in range(npt // zr):
            pltpu.sync_copy(zb, acc_sh.at[pl.ds(base + i * zr, zr)])

        pltpu.sync_copy(src_hbm.at[pl.ds(off, kmax + 2)], srcv)
        pltpu.sync_copy(dst_hbm.at[pl.ds(off, kmax)], dstv)
        pltpu.sync_copy(ewf_hbm.at[pl.ds(off * CH, kmax * CH)], ewv)

        def issue_gather(j, b):
            pltpu.async_copy(h2_hbm.at[srcv.at[j]], rowsb.at[b], gsems[b])

        def wait_gather(j, b):
            pltpu.make_async_copy(h2_hbm.at[srcv.at[j]], rowsb.at[b],
                                  gsems[b]).wait()

        def issue_scatter(j, b):
            pltpu.async_copy(rowsv.at[b], acc_sh.at[dstv.at[j]], ssems[b],
                             add=True)

        def wait_scatter(j, b):
            pltpu.make_async_copy(rowsv.at[b], acc_sh.at[dstv.at[j]],
                                  ssems[b]).wait()

        def scale(j, b):
            def grp(g, _):
                for kk in range(16):
                    le = g * 16 + kk
                    bi = jnp.full((16,), j * CH + le, jnp.int32)
                    wv = plsc.load_gather(ewv, [bi])
                    for q in range(h // 32):
                        mb = rowsb[b, le, pl.ds(q * 32, 32)]
                        w32 = plsc.bitcast(mb, jnp.int32)
                        va = plsc.bitcast(w32 << 16, jnp.float32)
                        vb = plsc.bitcast(
                            w32 & jnp.int32(-65536), jnp.float32)
                        rowsv[b, le, pl.ds(q * 32, 16)] = va * wv
                        rowsv[b, le, pl.ds(q * 32 + 16, 16)] = vb * wv
                return 0

            lax.fori_loop(0, CH // 16, grp, 0)

        issue_gather(0, 0)
        issue_gather(1, 1)
        wait_gather(0, 0)
        scale(0, 0)
        issue_scatter(0, 0)
        issue_gather(2, 2)
        wait_gather(1, 1)
        scale(1, 1)
        issue_scatter(1, 1)
        wait_scatter(0, 0)
        issue_gather(3, 0)

        def body(it, _):
            jj = 2 + it * NBUF
            for db in range(NBUF):
                j = jj + db
                b = (2 + db) % NBUF
                wait_gather(j, b)
                scale(j, b)
                issue_scatter(j, b)
                bn = (b + 2) % NBUF
                wait_scatter(j - 1, bn)
                issue_gather(j + 2, bn)
            return 0

        lax.fori_loop(0, (kd - 2) // NBUF, body, 0)
        wait_gather(kd, 2)
        wait_gather(kd + 1, 0)
        wait_scatter(kd - 1, 1)

        plsc.subcore_barrier()
        for i in range(npt // zr):
            pltpu.sync_copy(acc_sh.at[pl.ds(base + i * zr, zr)],
                            out_hbm.at[c, pl.ds(base + i * zr, zr)])

    return k


def _h2_body(x_ref, w1_ref, m1_ref, deg_ref, h2_ref, h2b_ref):
    wm = w1_ref[...] * m1_ref[...]
    hh = jnp.dot(x_ref[...], wm, preferred_element_type=jnp.float32)
    d = deg_ref[0, :] + deg_ref[1, :] + 1.0
    a = lax.rsqrt(d)
    h2 = hh * a[:, None]
    h2_ref[...] = h2
    hd = h2.shape[1]
    lcol = jax.lax.broadcasted_iota(jnp.int32, (hd, hd), 0)
    mcol = jax.lax.broadcasted_iota(jnp.int32, (hd, hd), 1)
    q, t = mcol // 32, mcol % 32
    want = q * 32 + (t % 2) * 16 + t // 2
    pmat = (lcol == want).astype(jnp.float32)
    h2b_ref[...] = jnp.dot(h2, pmat,
                           preferred_element_type=jnp.float32
                           ).astype(jnp.bfloat16)


def _final_body(acc_ref, h2_ref, deg_ref, b1_ref, w2_ref, m2_ref, b2_ref,
                out_ref):
    d = deg_ref[0, :] + deg_ref[1, :] + 1.0
    a = lax.rsqrt(d)
    tot = acc_ref[0] + acc_ref[1] + h2_ref[...]
    agg = tot * a[:, None] + b1_ref[0][None, :]
    hr = jnp.maximum(agg, 0.0)
    out_ref[...] = (jnp.dot(hr, w2_ref[...] * m2_ref[...],
                            preferred_element_type=jnp.float32)
                    + b2_ref[0][None, :])


def kernel(x, edge_index, edge_weight, W1, W1_mask, b1, W2, W2_mask, b2):
    n, f_in = x.shape
    h = W1.shape[1]
    c_out = W2.shape[1]
    e = edge_weight.shape[0]

    n_pad = ((n + NS * 16 - 1) // (NS * 16)) * (NS * 16)
    k0, k1 = _split(e)
    tot = NS * k0 + (NS - 1) * k1 + max(k0, k1) + 2
    e_pad = tot * CH

    src = edge_index[0].astype(jnp.int32)
    dst = edge_index[1].astype(jnp.int32)
    pad = e_pad - e
    src2 = jnp.concatenate([src, jnp.zeros((pad,), jnp.int32)]).reshape(tot, CH)
    dst2 = jnp.concatenate([dst, jnp.zeros((pad,), jnp.int32)]).reshape(tot, CH)
    ewp = jnp.concatenate([edge_weight, jnp.zeros((pad,), jnp.float32)])
    ew2 = ewp.reshape(tot, CH)

    deg2 = _deg_kernel(n_pad, k0, k1)(dst2, ew2)

    rb = 1024
    grid = -(-n // rb)
    h2 = pl.pallas_call(
        _h2_body,
        grid=(grid,),
        in_specs=[
            pl.BlockSpec((rb, f_in), lambda j: (j, 0)),
            pl.BlockSpec((f_in, h), lambda j: (0, 0)),
            pl.BlockSpec((f_in, h), lambda j: (0, 0)),
            pl.BlockSpec((NC, rb), lambda j: (0, j)),
        ],
        out_specs=[pl.BlockSpec((rb, h), lambda j: (j, 0)),
                   pl.BlockSpec((rb, h), lambda j: (j, 0))],
        out_shape=[jax.ShapeDtypeStruct((n, h), jnp.float32),
                   jax.ShapeDtypeStruct((n, h), jnp.bfloat16)],
    )(x, W1, W1_mask, deg2)
    h2, h2b = h2

    acc = _msg_kernel(n_pad, k0, k1, h)(src2, dst2, ewp, h2b)

    logits = pl.pallas_call(
        _final_body,
        grid=(grid,),
        in_specs=[
            pl.BlockSpec((NC, rb, h), lambda j: (0, j, 0)),
            pl.BlockSpec((rb, h), lambda j: (j, 0)),
            pl.BlockSpec((NC, rb), lambda j: (0, j)),
            pl.BlockSpec((1, h), lambda j: (0, 0)),
            pl.BlockSpec((h, c_out), lambda j: (0, 0)),
            pl.BlockSpec((h, c_out), lambda j: (0, 0)),
            pl.BlockSpec((1, c_out), lambda j: (0, 0)),
        ],
        out_specs=pl.BlockSpec((rb, c_out), lambda j: (j, 0)),
        out_shape=jax.ShapeDtypeStruct((n, c_out), jnp.float32),
    )(acc, h2, deg2, b1.reshape(1, h), W2, W2_mask, b2.reshape(1, c_out))

    return logits


# speedup vs baseline: 1.5573x; 1.2101x over previous
import functools

import jax
import jax.numpy as jnp
from jax import lax
from jax.experimental import pallas as pl
from jax.experimental.pallas import tpu as pltpu
from jax.experimental.pallas import tpu_sc as plsc

NC = 2
NS = 16
CH = 128
NBUF = 3
F0 = 0.627


def _split(e):
    tot_ch = -(-e // CH)
    t0 = F0 * tot_ch / NS
    k0 = max(5, min((v for v in range(int(t0) - 3, int(t0) + 5) if v % 3 == 2),
                    key=lambda v: abs(v - t0)))
    k1m = max(5, -(-(tot_ch - NS * k0) // NS))
    k1 = k1m + (2 - k1m) % 3
    return k0, k1


def _core_k_off(k0, k1):
    c = lax.axis_index("c")
    s = lax.axis_index("s")
    kd = jnp.where(c == 0, k0, k1)
    off = jnp.where(c == 0, s * k0, NS * k0 + s * k1)
    return c, s, kd, off


def _deg_kernel(n_pad, k0, k1):
    npt = n_pad // NS
    kmax = max(k0, k1)
    keq = -(-(NS * (k0 + k1)) // (NS * NC))

    @functools.partial(
        pl.kernel,
        out_type=jax.ShapeDtypeStruct((NC, n_pad), jnp.float32),
        mesh=plsc.VectorSubcoreMesh(core_axis_name="c", subcore_axis_name="s"),
        scratch_types=[
            pltpu.VMEM((kmax, CH), jnp.int32),
            pltpu.VMEM((kmax, CH), jnp.float32),
            pltpu.VMEM((npt,), jnp.float32),
            pltpu.VMEM_SHARED((n_pad,), jnp.float32),
        ],
        compiler_params=pltpu.CompilerParams(needs_layout_passes=False,
                                             use_tc_tiling_on_sc=False),
    )
    def k(dst_hbm, ew_hbm, out_hbm, dstv, ewv, zb, deg_sh):
        c = lax.axis_index("c")
        s = lax.axis_index("s")
        w = c * NS + s
        off = w * keq

        def zfill(i, _):
            zb[pl.ds(i * 16, 16)] = jnp.zeros((16,), jnp.float32)
            return 0

        lax.fori_loop(0, npt // 16, zfill, 0)
        pltpu.sync_copy(zb, deg_sh.at[pl.ds(s * npt, npt)])
        pltpu.sync_copy(dst_hbm.at[pl.ds(off, kmax)], dstv)
        pltpu.sync_copy(ew_hbm.at[pl.ds(off, kmax)], ewv)
        plsc.subcore_barrier()

        def chunk(j, _):
            pltpu.sync_copy(ewv.at[j], deg_sh.at[dstv.at[j]], add=True)
            return 0

        lax.fori_loop(0, keq, chunk, 0)
        plsc.subcore_barrier()
        pltpu.sync_copy(deg_sh.at[pl.ds(s * npt, npt)],
                        out_hbm.at[c, pl.ds(s * npt, npt)])

    return k


def _msg_kernel(n_pad, k0, k1, h):
    npt = n_pad // NS
    zr = 64
    kmax = max(k0, k1)

    @functools.partial(
        pl.kernel,
        out_type=jax.ShapeDtypeStruct((NC, n_pad, h), jnp.float32),
        mesh=plsc.VectorSubcoreMesh(core_axis_name="c", subcore_axis_name="s"),
        scratch_types=[
            pltpu.VMEM((kmax + 2, CH), jnp.int32),
            pltpu.VMEM((kmax, CH), jnp.int32),
            pltpu.VMEM((kmax * CH,), jnp.float32),
            pltpu.VMEM((NBUF, CH, h), jnp.float32),
            pltpu.VMEM((zr, h), jnp.float32),
            pltpu.VMEM_SHARED((n_pad, h), jnp.float32),
            [pltpu.SemaphoreType.DMA] * NBUF,
            [pltpu.SemaphoreType.DMA] * NBUF,
        ],
        compiler_params=pltpu.CompilerParams(needs_layout_passes=False,
                                             use_tc_tiling_on_sc=False),
    )
    def k(src_hbm, dst_hbm, ewf_hbm, h2_hbm, out_hbm,
          srcv, dstv, ewv, rowsv, zb, acc_sh, gsems, ssems):
        c, s, kd, off = _core_k_off(k0, k1)
        base = s * npt

        def zfill(i, _):
            for q in range(h // 16):
                zb[i, pl.ds(q * 16, 16)] = jnp.zeros((16,), jnp.float32)
            return 0

        lax.fori_loop(0, zr, zfill, 0)
        for i in range(npt // zr):
            pltpu.sync_copy(zb, acc_sh.at[pl.ds(base + i * zr, zr)])
        plsc.subcore_barrier()

        pltpu.sync_copy(src_hbm.at[pl.ds(off, kmax + 2)], srcv)
        pltpu.sync_copy(dst_hbm.at[pl.ds(off, kmax)], dstv)
        pltpu.sync_copy(ewf_hbm.at[pl.ds(off * CH, kmax * CH)], ewv)

        def issue_gather(j, b):
            pltpu.async_copy(h2_hbm.at[srcv.at[j]], rowsv.at[b], gsems[b])

        def wait_gather(j, b):
            pltpu.make_async_copy(h2_hbm.at[srcv.at[j]], rowsv.at[b],
                                  gsems[b]).wait()

        def issue_scatter(j, b):
            pltpu.async_copy(rowsv.at[b], acc_sh.at[dstv.at[j]], ssems[b],
                             add=True)

        def wait_scatter(j, b):
            pltpu.make_async_copy(rowsv.at[b], acc_sh.at[dstv.at[j]],
                                  ssems[b]).wait()

        def scale(j, b):
            def grp(g, _):
                for kk in range(16):
                    le = g * 16 + kk
                    bi = jnp.full((16,), j * CH + le, jnp.int32)
                    wv = plsc.load_gather(ewv, [bi])
                    for q in range(h // 16):
                        sl = pl.ds(q * 16, 16)
                        rowsv[b, le, sl] = rowsv[b, le, sl] * wv
                return 0

            lax.fori_loop(0, CH // 16, grp, 0)

        issue_gather(0, 0)
        issue_gather(1, 1)
        wait_gather(0, 0)
        scale(0, 0)
        issue_scatter(0, 0)
        issue_gather(2, 2)
        wait_gather(1, 1)
        scale(1, 1)
        issue_scatter(1, 1)
        wait_scatter(0, 0)
        issue_gather(3, 0)

        def body(it, _):
            jj = 2 + it * NBUF
            for db in range(NBUF):
                j = jj + db
                b = (2 + db) % NBUF
                wait_gather(j, b)
                scale(j, b)
                issue_scatter(j, b)
                bn = (b + 2) % NBUF
                wait_scatter(j - 1, bn)
                issue_gather(j + 2, bn)
            return 0

        lax.fori_loop(0, (kd - 2) // NBUF, body, 0)
        wait_gather(kd, 2)
        wait_gather(kd + 1, 0)
        wait_scatter(kd - 1, 1)

        plsc.subcore_barrier()
        for i in range(npt // zr):
            pltpu.sync_copy(acc_sh.at[pl.ds(base + i * zr, zr)],
                            out_hbm.at[c, pl.ds(base + i * zr, zr)])

    return k


def _h2_body(x_ref, w1_ref, m1_ref, deg_ref, h2_ref):
    wm = w1_ref[...] * m1_ref[...]
    hh = jnp.dot(x_ref[...], wm, preferred_element_type=jnp.float32)
    d = deg_ref[0, :] + deg_ref[1, :] + 1.0
    a = lax.rsqrt(d)
    h2_ref[...] = hh * a[:, None]


def _final_body(acc_ref, h2_ref, deg_ref, b1_ref, w2_ref, m2_ref, b2_ref,
                out_ref):
    d = deg_ref[0, :] + deg_ref[1, :] + 1.0
    a = lax.rsqrt(d)
    tot = acc_ref[0] + acc_ref[1] + h2_ref[...]
    agg = tot * a[:, None] + b1_ref[0][None, :]
    hr = jnp.maximum(agg, 0.0)
    out_ref[...] = (jnp.dot(hr, w2_ref[...] * m2_ref[...],
                            preferred_element_type=jnp.float32)
                    + b2_ref[0][None, :])


def kernel(x, edge_index, edge_weight, W1, W1_mask, b1, W2, W2_mask, b2):
    n, f_in = x.shape
    h = W1.shape[1]
    c_out = W2.shape[1]
    e = edge_weight.shape[0]

    n_pad = ((n + NS * 16 - 1) // (NS * 16)) * (NS * 16)
    k0, k1 = _split(e)
    keq = -(-(NS * (k0 + k1)) // (NS * NC))
    tot = max(NS * k0 + (NS - 1) * k1 + max(k0, k1) + 2,
              (NC * NS - 1) * keq + max(k0, k1))
    e_pad = tot * CH

    src = edge_index[0].astype(jnp.int32)
    dst = edge_index[1].astype(jnp.int32)
    pad = e_pad - e
    src2 = jnp.concatenate([src, jnp.zeros((pad,), jnp.int32)]).reshape(tot, CH)
    dst2 = jnp.concatenate([dst, jnp.zeros((pad,), jnp.int32)]).reshape(tot, CH)
    ewp = jnp.concatenate([edge_weight, jnp.zeros((pad,), jnp.float32)])
    ew2 = ewp.reshape(tot, CH)

    deg2 = _deg_kernel(n_pad, k0, k1)(dst2, ew2)

    rb = 1024
    grid = -(-n // rb)
    h2 = pl.pallas_call(
        _h2_body,
        grid=(grid,),
        in_specs=[
            pl.BlockSpec((rb, f_in), lambda j: (j, 0)),
            pl.BlockSpec((f_in, h), lambda j: (0, 0)),
            pl.BlockSpec((f_in, h), lambda j: (0, 0)),
            pl.BlockSpec((NC, rb), lambda j: (0, j)),
        ],
        out_specs=pl.BlockSpec((rb, h), lambda j: (j, 0)),
        out_shape=jax.ShapeDtypeStruct((n, h), jnp.float32),
    )(x, W1, W1_mask, deg2)

    acc = _msg_kernel(n_pad, k0, k1, h)(src2, dst2, ewp, h2)

    logits = pl.pallas_call(
        _final_body,
        grid=(grid,),
        in_specs=[
            pl.BlockSpec((NC, rb, h), lambda j: (0, j, 0)),
            pl.BlockSpec((rb, h), lambda j: (j, 0)),
            pl.BlockSpec((NC, rb), lambda j: (0, j)),
            pl.BlockSpec((1, h), lambda j: (0, 0)),
            pl.BlockSpec((h, c_out), lambda j: (0, 0)),
            pl.BlockSpec((h, c_out), lambda j: (0, 0)),
            pl.BlockSpec((1, c_out), lambda j: (0, 0)),
        ],
        out_specs=pl.BlockSpec((rb, c_out), lambda j: (j, 0)),
        out_shape=jax.ShapeDtypeStruct((n, c_out), jnp.float32),
    )(acc, h2, deg2, b1.reshape(1, h), W2, W2_mask, b2.reshape(1, c_out))

    return logits
